# Initial kernel scaffold; baseline (speedup 1.0000x reference)
import jax
import jax.numpy as jnp
from jax.experimental import pallas as pl


def kernel(x, edge_index, edge_attr, batch, atom_emb, bond_emb, eps, W1, b1, g1, beta1, W2, b2, g2, beta2, lin_W, lin_b):
    raise NotImplementedError("write your pallas kernel here")


# speedup vs baseline: 2.7645x; 2.7645x over previous
import functools

import jax
import jax.numpy as jnp
from jax import lax
from jax.experimental import pallas as pl
from jax.experimental.pallas import tpu as pltpu
from jax.experimental.pallas import tpu_sc as plsc

N = 10000
E = 320000
H = 128
OUT = 128
G = 256
L = 3

NC = 2
NS = 16
NW = NC * NS
CHUNK = 128
NCHUNK = 79
E2 = NW * NCHUNK * CHUNK

NODE_CHUNKS = 3
N2 = NW * NODE_CHUNKS * CHUNK
ROWS_PER_TILE = 632
N3 = NS * ROWS_PER_TILE

_mesh = plsc.VectorSubcoreMesh(core_axis_name="c", subcore_axis_name="s")


def _zero_rows(buf, nrows):
    zeros16 = jnp.zeros((1, 16), jnp.float32)

    @pl.loop(0, nrows)
    def _(r):
        for g in range(H // 16):
            buf[pl.ds(r, 1), pl.ds(g * 16, 16)] = zeros16


def _accum_rows(acc, src, nrows):
    @pl.loop(0, nrows)
    def _(r):
        for g in range(H // 16):
            sl = (pl.ds(r, 1), pl.ds(g * 16, 16))
            acc[sl] = acc[sl] + src[sl]


@functools.partial(
    pl.kernel,
    out_type=jax.ShapeDtypeStruct((N2, H), jnp.float32),
    mesh=_mesh,
    scratch_types=[
        pltpu.VMEM((9 * CHUNK,), jnp.int32),
        pltpu.VMEM((CHUNK, H), jnp.float32),
        pltpu.VMEM((CHUNK, H), jnp.float32),
    ],
)
def _encoder_kernel(xt_hbm, e0, e1, e2, e3, e4, e5, e6, e7, e8, o_hbm,
                    xv, acc, tmp):
    embs = (e0, e1, e2, e3, e4, e5, e6, e7, e8)
    w = lax.axis_index("c") * NS + lax.axis_index("s")
    for k in range(NODE_CHUNKS):
        pltpu.sync_copy(
            xt_hbm.at[pl.ds((w * NODE_CHUNKS + k) * 9 * CHUNK, 9 * CHUNK)], xv)
        pltpu.sync_copy(embs[0].at[xv.at[pl.ds(0, CHUNK)]], acc)
        for f in range(1, 9):
            pltpu.sync_copy(embs[f].at[xv.at[pl.ds(f * CHUNK, CHUNK)]], tmp)
            _accum_rows(acc, tmp, CHUNK)
        pltpu.sync_copy(acc, o_hbm.at[pl.ds(w * (NODE_CHUNKS * CHUNK) + k * CHUNK, CHUNK)])


@functools.partial(
    pl.kernel,
    out_type=[
        jax.ShapeDtypeStruct((N3, H), jnp.float32),
        jax.ShapeDtypeStruct((N3, H), jnp.float32),
    ],
    mesh=_mesh,
    scratch_types=[
        pltpu.VMEM((2 * CHUNK,), jnp.int32),
        pltpu.VMEM((CHUNK,), jnp.int32),
        pltpu.VMEM((CHUNK, H), jnp.float32),
        pltpu.VMEM((CHUNK, H), jnp.float32),
        pltpu.VMEM_SHARED((N3, H), jnp.float32),
    ],
)
def _edge_kernel(h_hbm, tab_hbm, idx_hbm, out0, out1,
                 sc_v, dst_v, hbuf, ebuf, aggr):
    cid = lax.axis_index("c")
    sid = lax.axis_index("s")
    w = cid * NS + sid

    _zero_rows(hbuf, CHUNK)
    base = sid * ROWS_PER_TILE
    nfull = ROWS_PER_TILE // CHUNK
    for k in range(nfull):
        pltpu.sync_copy(hbuf, aggr.at[pl.ds(base + k * CHUNK, CHUNK)])
    rem = ROWS_PER_TILE - nfull * CHUNK
    if rem:
        pltpu.sync_copy(hbuf.at[pl.ds(0, rem)],
                        aggr.at[pl.ds(base + nfull * CHUNK, rem)])
    plsc.subcore_barrier()

    @pl.loop(0, NCHUNK)
    def _(j):
        base_i = (w * NCHUNK + j) * 3 * CHUNK
        pltpu.sync_copy(idx_hbm.at[pl.ds(base_i, 2 * CHUNK)], sc_v)
        pltpu.sync_copy(idx_hbm.at[pl.ds(base_i + 2 * CHUNK, CHUNK)], dst_v)
        pltpu.sync_copy(h_hbm.at[sc_v.at[pl.ds(0, CHUNK)]], hbuf)
        pltpu.sync_copy(tab_hbm.at[sc_v.at[pl.ds(CHUNK, CHUNK)]], ebuf)

        @pl.loop(0, CHUNK)
        def _(r):
            for g in range(H // 16):
                sl = (pl.ds(r, 1), pl.ds(g * 16, 16))
                hbuf[sl] = jnp.maximum(hbuf[sl] + ebuf[sl], 0.0)

        pltpu.sync_copy(hbuf, aggr.at[dst_v], add=True)

    plsc.subcore_barrier()

    slab = pl.ds(base, ROWS_PER_TILE)

    @pl.when(cid == 0)
    def _():
        pltpu.sync_copy(aggr.at[slab], out0.at[slab])

    @pl.when(cid == 1)
    def _():
        pltpu.sync_copy(aggr.at[slab], out1.at[slab])


def _tab_body(bond_ref, o_ref):
    for i0 in range(5):
        b0 = bond_ref[0, pl.ds(i0, 1), :]
        for i1 in range(5):
            b01 = b0 + bond_ref[1, pl.ds(i1, 1), :]
            for i2 in range(5):
                o_ref[pl.ds(i0 * 25 + i1 * 5 + i2, 1), :] = (
                    b01 + bond_ref[2, pl.ds(i2, 1), :])
    o_ref[pl.ds(125, 1), :] = jnp.full((1, H), -1e30, jnp.float32)
    o_ref[pl.ds(126, 2), :] = jnp.zeros((2, H), jnp.float32)


def _tab_call(bond_emb):
    return pl.pallas_call(
        _tab_body,
        out_shape=jax.ShapeDtypeStruct((128, H), jnp.float32),
    )(bond_emb)


def _mlp_body(h_ref, a0_ref, a1_ref, epsb_ref, w1_ref, b1_ref, g1_ref,
              be1_ref, w2_ref, b2_ref, g2_ref, be2_ref, o_ref):
    z = h_ref[...] * epsb_ref[...] + (a0_ref[...] + a1_ref[...])
    z1 = jnp.dot(z, w1_ref[...], preferred_element_type=jnp.float32)
    z1 = z1 + b1_ref[...]
    m1 = jnp.mean(z1, axis=0, keepdims=True)
    v1 = jnp.mean((z1 - m1) ** 2, axis=0, keepdims=True)
    z1 = (z1 - m1) / jnp.sqrt(v1 + 1e-5) * g1_ref[...] + be1_ref[...]
    z1 = jnp.maximum(z1, 0.0)
    z2 = jnp.dot(z1, w2_ref[...], preferred_element_type=jnp.float32)
    z2 = z2 + b2_ref[...]
    m2 = jnp.mean(z2, axis=0, keepdims=True)
    v2 = jnp.mean((z2 - m2) ** 2, axis=0, keepdims=True)
    z2 = (z2 - m2) / jnp.sqrt(v2 + 1e-5) * g2_ref[...] + be2_ref[...]
    o_ref[...] = jnp.maximum(z2, 0.0)


def _mlp_call(h, a0, a1, epsb, w1, b1, g1, be1, w2, b2, g2, be2):
    return pl.pallas_call(
        _mlp_body,
        out_shape=jax.ShapeDtypeStruct((N, H), jnp.float32),
    )(h, a0, a1, epsb, w1, b1, g1, be1, w2, b2, g2, be2)


def _pool_body(h_ref, batch_ref, lw_ref, lb_ref, o_ref):
    onehot = (batch_ref[...] == lax.broadcasted_iota(jnp.int32, (N, G), 1)
              ).astype(jnp.float32)
    dn = (((0,), (0,)), ((), ()))
    sums = lax.dot_general(onehot, h_ref[...], dn,
                           preferred_element_type=jnp.float32)
    cnt = lax.dot_general(onehot, jnp.ones((N, 1), jnp.float32), dn,
                          preferred_element_type=jnp.float32)
    pooled = sums / jnp.maximum(cnt, 1.0)
    o_ref[...] = jnp.dot(pooled, lw_ref[...],
                         preferred_element_type=jnp.float32) + lb_ref[...]


def _pool_call(h, batch2d, lin_W, lin_b2d):
    return pl.pallas_call(
        _pool_body,
        out_shape=jax.ShapeDtypeStruct((G, OUT), jnp.float32),
    )(h, batch2d, lin_W, lin_b2d)


def kernel(x, edge_index, edge_attr, batch, atom_emb, bond_emb, eps,
           W1, b1, g1, beta1, W2, b2, g2, beta2, lin_W, lin_b):
    src = edge_index[0]
    dst = edge_index[1]
    c = edge_attr[:, 0] * 25 + edge_attr[:, 1] * 5 + edge_attr[:, 2]
    pad = E2 - E
    srcp = jnp.concatenate([src, jnp.zeros((pad,), jnp.int32)]
                           ).reshape(NW, NCHUNK, 1, CHUNK)
    dstp = jnp.concatenate([dst, jnp.zeros((pad,), jnp.int32)]
                           ).reshape(NW, NCHUNK, 1, CHUNK)
    cp = jnp.concatenate([c, jnp.full((pad,), 125, jnp.int32)]
                         ).reshape(NW, NCHUNK, 1, CHUNK)
    idxp = jnp.concatenate([srcp, cp, dstp], axis=2).reshape(-1)
    xt = jnp.pad(x, ((0, N2 - N), (0, 0))).T
    xt = (xt.reshape(9, NW, NODE_CHUNKS, CHUNK).transpose(1, 2, 0, 3)
          .reshape(-1))
    batch2d = batch.reshape(N, 1)

    tab = _tab_call(bond_emb)
    h = _encoder_kernel(xt, *(atom_emb[f] for f in range(9)))[:N]
    for l in range(L):
        a0, a1 = _edge_kernel(h, tab, idxp)
        a0, a1 = a0[:N], a1[:N]
        epsb = jnp.broadcast_to(1.0 + eps[l], (1, H)).astype(jnp.float32)
        h = _mlp_call(h, a0, a1, epsb, W1[l], b1[l].reshape(1, 2 * H),
                      g1[l].reshape(1, 2 * H), beta1[l].reshape(1, 2 * H),
                      W2[l], b2[l].reshape(1, H), g2[l].reshape(1, H),
                      beta2[l].reshape(1, H))
    return _pool_call(h, batch2d, lin_W, lin_b.reshape(1, OUT))


# speedup vs baseline: 3.5064x; 1.2684x over previous
import functools

import jax
import jax.numpy as jnp
from jax import lax
from jax.experimental import pallas as pl
from jax.experimental.pallas import tpu as pltpu
from jax.experimental.pallas import tpu_sc as plsc

N = 10000
E = 320000
H = 128
OUT = 128
G = 256
L = 3

NC = 2
NS = 16
NW = NC * NS
CHUNK = 128
NCHUNK = 80
E2 = NW * NCHUNK * CHUNK

NODE_CHUNKS = 3
N2 = NW * NODE_CHUNKS * CHUNK
ROWS_PER_TILE = 632
N3 = NS * ROWS_PER_TILE

_mesh = plsc.VectorSubcoreMesh(core_axis_name="c", subcore_axis_name="s")


def _zero_rows(buf, nrows):
    zeros16 = jnp.zeros((1, 16), jnp.float32)

    @pl.loop(0, nrows)
    def _(r):
        for g in range(H // 16):
            buf[pl.ds(r, 1), pl.ds(g * 16, 16)] = zeros16


def _accum_rows(acc, src, nrows):
    @pl.loop(0, nrows)
    def _(r):
        for g in range(H // 16):
            sl = (pl.ds(r, 1), pl.ds(g * 16, 16))
            acc[sl] = acc[sl] + src[sl]


@functools.partial(
    pl.kernel,
    out_type=jax.ShapeDtypeStruct((N2, H), jnp.float32),
    mesh=_mesh,
    scratch_types=[
        pltpu.VMEM((NODE_CHUNKS * 9 * CHUNK,), jnp.int32),
        pltpu.VMEM((CHUNK, H), jnp.float32),
        pltpu.VMEM((CHUNK, H), jnp.float32),
        pltpu.VMEM((CHUNK, H), jnp.float32),
        pltpu.VMEM((CHUNK, H), jnp.float32),
        pltpu.SemaphoreType.DMA,
        pltpu.SemaphoreType.DMA,
        pltpu.SemaphoreType.DMA,
        pltpu.SemaphoreType.DMA,
        pltpu.SemaphoreType.DMA,
        pltpu.SemaphoreType.DMA,
    ],
)
def _encoder_kernel(xt_hbm, e0, e1, e2, e3, e4, e5, e6, e7, e8, o_hbm,
                    xv, acc0, acc1, tmp0, tmp1, aS0, aS1, tS0, tS1, oS0, oS1):
    embs = (e0, e1, e2, e3, e4, e5, e6, e7, e8)
    accs = (acc0, acc1)
    asem = (aS0, aS1)
    tmps = (tmp0, tmp1)
    tsem = (tS0, tS1)
    osem = (oS0, oS1)
    w = lax.axis_index("c") * NS + lax.axis_index("s")
    nidx = NODE_CHUNKS * 9 * CHUNK
    pltpu.sync_copy(xt_hbm.at[pl.ds(w * nidx, nidx)], xv)

    def idx(k, f):
        return xv.at[pl.ds((k * 9 + f) * CHUNK, CHUNK)]

    def wait_rows(buf, sem):
        pltpu.make_async_copy(e0.at[pl.ds(0, CHUNK)], buf, sem).wait()

    for k in range(NODE_CHUNKS):
        a = k % 2
        if k >= 2:
            pltpu.make_async_copy(accs[a], o_hbm.at[pl.ds(0, CHUNK)],
                                  osem[a]).wait()
        pltpu.async_copy(embs[0].at[idx(k, 0)], accs[a], asem[a])
        pltpu.async_copy(embs[1].at[idx(k, 1)], tmps[0], tsem[0])
        wait_rows(accs[a], asem[a])
        for f in range(1, 9):
            if f + 1 <= 8:
                pltpu.async_copy(embs[f + 1].at[idx(k, f + 1)],
                                 tmps[f % 2], tsem[f % 2])
            wait_rows(tmps[(f - 1) % 2], tsem[(f - 1) % 2])
            _accum_rows(accs[a], tmps[(f - 1) % 2], CHUNK)
        pltpu.async_copy(
            accs[a],
            o_hbm.at[pl.ds(w * (NODE_CHUNKS * CHUNK) + k * CHUNK, CHUNK)],
            osem[a])
    for k in range(max(0, NODE_CHUNKS - 2), NODE_CHUNKS):
        pltpu.make_async_copy(accs[k % 2], o_hbm.at[pl.ds(0, CHUNK)],
                              osem[k % 2]).wait()


def _msg_compute(hb, tab_v, cs):
    @pl.loop(0, CHUNK // 16)
    def _(q):
        cvec = cs[pl.ds(q * 16, 16)]
        for k in range(16):
            cval = cvec[k]
            rsl = pl.ds(q * 16 + k, 1)
            for g in range(H // 16):
                csl = pl.ds(g * 16, 16)
                sl = (rsl, csl)
                hb[sl] = jnp.maximum(hb[sl] + tab_v[pl.ds(cval, 1), csl],
                                     0.0)


@functools.partial(
    pl.kernel,
    out_type=[
        jax.ShapeDtypeStruct((N3, H), jnp.float32),
        jax.ShapeDtypeStruct((N3, H), jnp.float32),
    ],
    mesh=_mesh,
    scratch_types=[
        pltpu.VMEM((CHUNK,), jnp.int32),
        pltpu.VMEM((CHUNK,), jnp.int32),
        pltpu.VMEM((CHUNK,), jnp.int32),
        pltpu.VMEM((CHUNK,), jnp.int32),
        pltpu.VMEM((CHUNK,), jnp.int32),
        pltpu.VMEM((CHUNK,), jnp.int32),
        pltpu.VMEM((128, H), jnp.float32),
        pltpu.VMEM((CHUNK, H), jnp.float32),
        pltpu.VMEM((CHUNK, H), jnp.float32),
        pltpu.VMEM_SHARED((N3, H), jnp.float32),
        pltpu.SemaphoreType.DMA,
        pltpu.SemaphoreType.DMA,
        pltpu.SemaphoreType.DMA,
        pltpu.SemaphoreType.DMA,
        pltpu.SemaphoreType.DMA,
        pltpu.SemaphoreType.DMA,
        pltpu.SemaphoreType.DMA,
        pltpu.SemaphoreType.DMA,
        pltpu.SemaphoreType.DMA,
        pltpu.SemaphoreType.DMA,
    ],
)
def _edge_kernel(h_hbm, tab_hbm, src_hbm, c_hbm, dst_hbm, out0, out1,
                 sv0, sv1, dv0, dv1, cv0, cv1, tab_v, hb0, hb1, aggr,
                 g0, g1, s0, s1, dI0, dI1, cI0, cI1, rI0, rI1):
    cid = lax.axis_index("c")
    sid = lax.axis_index("s")
    w = cid * NS + sid
    svs = (sv0, sv1)
    dvs = (dv0, dv1)
    cvs = (cv0, cv1)
    hbs = (hb0, hb1)
    gsem = (g0, g1)
    ssem = (s0, s1)
    dsem = (dI0, dI1)
    csem = (cI0, cI1)
    rsem = (rI0, rI1)
    ebase = w * NCHUNK * CHUNK

    pltpu.sync_copy(tab_hbm, tab_v)

    _zero_rows(hb0, CHUNK)
    base = sid * ROWS_PER_TILE
    nfull = ROWS_PER_TILE // CHUNK
    for k in range(nfull):
        pltpu.sync_copy(hb0, aggr.at[pl.ds(base + k * CHUNK, CHUNK)])
    rem = ROWS_PER_TILE - nfull * CHUNK
    if rem:
        pltpu.sync_copy(hb0.at[pl.ds(0, rem)],
                        aggr.at[pl.ds(base + nfull * CHUNK, rem)])
    plsc.subcore_barrier()

    def src_load(j, k):
        pltpu.async_copy(src_hbm.at[pl.ds(ebase + j * CHUNK, CHUNK)],
                         svs[k], rsem[k])

    def c_load(j, k):
        pltpu.async_copy(c_hbm.at[pl.ds(ebase + j * CHUNK, CHUNK)],
                         cvs[k], csem[k])

    def dst_load(j, k):
        pltpu.async_copy(dst_hbm.at[pl.ds(ebase + j * CHUNK, CHUNK)],
                         dvs[k], dsem[k])

    def gather(j, p):
        pltpu.async_copy(h_hbm.at[svs[p]], hbs[p], gsem[p])

    def wait_src(k):
        pltpu.make_async_copy(src_hbm.at[pl.ds(0, CHUNK)], svs[k],
                              rsem[k]).wait()

    def wait_c(k):
        pltpu.make_async_copy(c_hbm.at[pl.ds(0, CHUNK)], cvs[k],
                              csem[k]).wait()

    def wait_dst(k):
        pltpu.make_async_copy(dst_hbm.at[pl.ds(0, CHUNK)], dvs[k],
                              dsem[k]).wait()

    def wait_gather(p):
        pltpu.make_async_copy(h_hbm.at[pl.ds(0, CHUNK)], hbs[p],
                              gsem[p]).wait()

    def wait_scatter(p):
        pltpu.make_async_copy(hbs[p], aggr.at[pl.ds(0, CHUNK)],
                              ssem[p]).wait()

    pltpu.sync_copy(src_hbm.at[pl.ds(ebase, CHUNK)], sv0)
    c_load(0, 0)
    dst_load(0, 0)
    gather(0, 0)
    src_load(1, 1)
    c_load(1, 1)

    NT = NCHUNK // 2

    @pl.loop(0, NT)
    def _(t):
        for u in range(2):
            p = u
            pn = (u + 1) % 2
            if u == 0:
                @pl.when(t > 0)
                def _():
                    wait_scatter(pn)
            else:
                wait_scatter(pn)
            if u == 0:
                dst_load(2 * t + 1, pn)
            else:
                @pl.when(t < NT - 1)
                def _():
                    dst_load(2 * t + 2, pn)
            if u == 0:
                wait_src(pn)
                gather(2 * t + 1, pn)
            else:
                @pl.when(t < NT - 1)
                def _():
                    wait_src(pn)
                    gather(2 * t + 2, pn)
            wait_gather(p)
            wait_c(p)
            _msg_compute(hbs[p], tab_v, cvs[p])
            wait_dst(p)
            pltpu.async_copy(hbs[p], aggr.at[dvs[p]], ssem[p], add=True)
            if u == 0:
                @pl.when(t < NT - 1)
                def _():
                    src_load(2 * t + 2, p)
                    c_load(2 * t + 2, p)
            else:
                @pl.when(t < NT - 1)
                def _():
                    src_load(2 * t + 3, p)
                    c_load(2 * t + 3, p)

    wait_scatter(1)
    plsc.subcore_barrier()

    slab = pl.ds(base, ROWS_PER_TILE)

    @pl.when(cid == 0)
    def _():
        pltpu.sync_copy(aggr.at[slab], out0.at[slab])

    @pl.when(cid == 1)
    def _():
        pltpu.sync_copy(aggr.at[slab], out1.at[slab])


def _tab_body(bond_ref, o_ref):
    for i0 in range(5):
        b0 = bond_ref[0, pl.ds(i0, 1), :]
        for i1 in range(5):
            b01 = b0 + bond_ref[1, pl.ds(i1, 1), :]
            for i2 in range(5):
                o_ref[pl.ds(i0 * 25 + i1 * 5 + i2, 1), :] = (
                    b01 + bond_ref[2, pl.ds(i2, 1), :])
    o_ref[pl.ds(125, 1), :] = jnp.full((1, H), -1e30, jnp.float32)
    o_ref[pl.ds(126, 2), :] = jnp.zeros((2, H), jnp.float32)


def _tab_call(bond_emb):
    return pl.pallas_call(
        _tab_body,
        out_shape=jax.ShapeDtypeStruct((128, H), jnp.float32),
    )(bond_emb)


def _mlp_body(h_ref, a0_ref, a1_ref, epsb_ref, w1_ref, b1_ref, g1_ref,
              be1_ref, w2_ref, b2_ref, g2_ref, be2_ref, o_ref):
    z = h_ref[...] * epsb_ref[...] + (a0_ref[...] + a1_ref[...])
    z1 = jnp.dot(z, w1_ref[...], preferred_element_type=jnp.float32)
    z1 = z1 + b1_ref[...]
    m1 = jnp.mean(z1, axis=0, keepdims=True)
    v1 = jnp.mean((z1 - m1) ** 2, axis=0, keepdims=True)
    z1 = (z1 - m1) / jnp.sqrt(v1 + 1e-5) * g1_ref[...] + be1_ref[...]
    z1 = jnp.maximum(z1, 0.0)
    z2 = jnp.dot(z1, w2_ref[...], preferred_element_type=jnp.float32)
    z2 = z2 + b2_ref[...]
    m2 = jnp.mean(z2, axis=0, keepdims=True)
    v2 = jnp.mean((z2 - m2) ** 2, axis=0, keepdims=True)
    z2 = (z2 - m2) / jnp.sqrt(v2 + 1e-5) * g2_ref[...] + be2_ref[...]
    o_ref[...] = jnp.maximum(z2, 0.0)


def _mlp_call(h, a0, a1, epsb, w1, b1, g1, be1, w2, b2, g2, be2):
    return pl.pallas_call(
        _mlp_body,
        out_shape=jax.ShapeDtypeStruct((N, H), jnp.float32),
    )(h, a0, a1, epsb, w1, b1, g1, be1, w2, b2, g2, be2)


def _pool_body(h_ref, batch_ref, lw_ref, lb_ref, o_ref):
    onehot = (batch_ref[...] == lax.broadcasted_iota(jnp.int32, (N, G), 1)
              ).astype(jnp.float32)
    dn = (((0,), (0,)), ((), ()))
    sums = lax.dot_general(onehot, h_ref[...], dn,
                           preferred_element_type=jnp.float32)
    cnt = lax.dot_general(onehot, jnp.ones((N, 1), jnp.float32), dn,
                          preferred_element_type=jnp.float32)
    pooled = sums / jnp.maximum(cnt, 1.0)
    o_ref[...] = jnp.dot(pooled, lw_ref[...],
                         preferred_element_type=jnp.float32) + lb_ref[...]


def _pool_call(h, batch2d, lin_W, lin_b2d):
    return pl.pallas_call(
        _pool_body,
        out_shape=jax.ShapeDtypeStruct((G, OUT), jnp.float32),
    )(h, batch2d, lin_W, lin_b2d)


def kernel(x, edge_index, edge_attr, batch, atom_emb, bond_emb, eps,
           W1, b1, g1, beta1, W2, b2, g2, beta2, lin_W, lin_b):
    src = edge_index[0]
    dst = edge_index[1]
    c = edge_attr[:, 0] * 25 + edge_attr[:, 1] * 5 + edge_attr[:, 2]
    pad = E2 - E
    srcp = jnp.concatenate([src, jnp.zeros((pad,), jnp.int32)])
    dstp = jnp.concatenate([dst, jnp.zeros((pad,), jnp.int32)])
    cp = jnp.concatenate([c, jnp.full((pad,), 125, jnp.int32)])
    xt = jnp.pad(x, ((0, N2 - N), (0, 0))).T
    xt = (xt.reshape(9, NW, NODE_CHUNKS, CHUNK).transpose(1, 2, 0, 3)
          .reshape(-1))
    batch2d = batch.reshape(N, 1)

    tab = _tab_call(bond_emb)
    h = _encoder_kernel(xt, *(atom_emb[f] for f in range(9)))[:N]
    for l in range(L):
        a0, a1 = _edge_kernel(h, tab, srcp, cp, dstp)
        a0, a1 = a0[:N], a1[:N]
        epsb = jnp.broadcast_to(1.0 + eps[l], (1, H)).astype(jnp.float32)
        h = _mlp_call(h, a0, a1, epsb, W1[l], b1[l].reshape(1, 2 * H),
                      g1[l].reshape(1, 2 * H), beta1[l].reshape(1, 2 * H),
                      W2[l], b2[l].reshape(1, H), g2[l].reshape(1, H),
                      beta2[l].reshape(1, H))
    return _pool_call(h, batch2d, lin_W, lin_b.reshape(1, OUT))


# speedup vs baseline: 3.9905x; 1.1381x over previous
import functools

import jax
import jax.numpy as jnp
from jax import lax
from jax.experimental import pallas as pl
from jax.experimental.pallas import tpu as pltpu
from jax.experimental.pallas import tpu_sc as plsc

N = 10000
E = 320000
H = 128
OUT = 128
G = 256
L = 3

NC = 2
NS = 16
NW = NC * NS
CHUNK = 128
NCHUNK = 80
E2 = NW * NCHUNK * CHUNK

NODE_CHUNKS = 3
N2 = NW * NODE_CHUNKS * CHUNK
ROWS_PER_TILE = 632
N3 = NS * ROWS_PER_TILE

_mesh = plsc.VectorSubcoreMesh(core_axis_name="c", subcore_axis_name="s")


def _zero_rows(buf, nrows):
    zeros16 = jnp.zeros((1, 16), jnp.float32)

    @pl.loop(0, nrows)
    def _(r):
        for g in range(H // 16):
            buf[pl.ds(r, 1), pl.ds(g * 16, 16)] = zeros16


def _accum_rows(acc, src, nrows):
    @pl.loop(0, nrows)
    def _(r):
        for g in range(H // 16):
            sl = (pl.ds(r, 1), pl.ds(g * 16, 16))
            acc[sl] = acc[sl] + src[sl]


@functools.partial(
    pl.kernel,
    out_type=jax.ShapeDtypeStruct((N2, H), jnp.float32),
    mesh=_mesh,
    scratch_types=[
        pltpu.VMEM((NODE_CHUNKS * 9 * CHUNK,), jnp.int32),
        pltpu.VMEM((CHUNK, H), jnp.float32),
        pltpu.VMEM((CHUNK, H), jnp.float32),
        pltpu.VMEM((CHUNK, H), jnp.float32),
        pltpu.VMEM((CHUNK, H), jnp.float32),
        pltpu.SemaphoreType.DMA,
        pltpu.SemaphoreType.DMA,
        pltpu.SemaphoreType.DMA,
        pltpu.SemaphoreType.DMA,
        pltpu.SemaphoreType.DMA,
        pltpu.SemaphoreType.DMA,
    ],
)
def _encoder_kernel(xt_hbm, e0, e1, e2, e3, e4, e5, e6, e7, e8, o_hbm,
                    xv, acc0, acc1, tmp0, tmp1, aS0, aS1, tS0, tS1, oS0, oS1):
    embs = (e0, e1, e2, e3, e4, e5, e6, e7, e8)
    accs = (acc0, acc1)
    asem = (aS0, aS1)
    tmps = (tmp0, tmp1)
    tsem = (tS0, tS1)
    osem = (oS0, oS1)
    w = lax.axis_index("c") * NS + lax.axis_index("s")
    nidx = NODE_CHUNKS * 9 * CHUNK
    pltpu.sync_copy(xt_hbm.at[pl.ds(w * nidx, nidx)], xv)

    def idx(k, f):
        return xv.at[pl.ds((k * 9 + f) * CHUNK, CHUNK)]

    def wait_rows(buf, sem):
        pltpu.make_async_copy(e0.at[pl.ds(0, CHUNK)], buf, sem).wait()

    for k in range(NODE_CHUNKS):
        a = k % 2
        if k >= 2:
            pltpu.make_async_copy(accs[a], o_hbm.at[pl.ds(0, CHUNK)],
                                  osem[a]).wait()
        pltpu.async_copy(embs[0].at[idx(k, 0)], accs[a], asem[a])
        pltpu.async_copy(embs[1].at[idx(k, 1)], tmps[0], tsem[0])
        wait_rows(accs[a], asem[a])
        for f in range(1, 9):
            if f + 1 <= 8:
                pltpu.async_copy(embs[f + 1].at[idx(k, f + 1)],
                                 tmps[f % 2], tsem[f % 2])
            wait_rows(tmps[(f - 1) % 2], tsem[(f - 1) % 2])
            _accum_rows(accs[a], tmps[(f - 1) % 2], CHUNK)
        pltpu.async_copy(
            accs[a],
            o_hbm.at[pl.ds(w * (NODE_CHUNKS * CHUNK) + k * CHUNK, CHUNK)],
            osem[a])
    for k in range(max(0, NODE_CHUNKS - 2), NODE_CHUNKS):
        pltpu.make_async_copy(accs[k % 2], o_hbm.at[pl.ds(0, CHUNK)],
                              osem[k % 2]).wait()


def _msg_compute(hb, tab_v, cs):
    ng = H // 16

    @pl.loop(0, CHUNK // 16)
    def _(q):
        cvec = cs[pl.ds(q * 16, 16)]
        for k in range(16):
            cval = cvec[k]
            rsl = pl.ds(q * 16 + k, 1)
            trow = pl.ds(cval, 1)
            hv = [hb[rsl, pl.ds(g * 16, 16)] for g in range(ng)]
            tv = [tab_v[trow, pl.ds(g * 16, 16)] for g in range(ng)]
            res = [jnp.maximum(hv[g] + tv[g], 0.0) for g in range(ng)]
            for g in range(ng):
                hb[rsl, pl.ds(g * 16, 16)] = res[g]


@functools.partial(
    pl.kernel,
    out_type=[
        jax.ShapeDtypeStruct((N3, H), jnp.float32),
        jax.ShapeDtypeStruct((N3, H), jnp.float32),
    ],
    mesh=_mesh,
    scratch_types=[
        pltpu.VMEM((CHUNK,), jnp.int32),
        pltpu.VMEM((CHUNK,), jnp.int32),
        pltpu.VMEM((CHUNK,), jnp.int32),
        pltpu.VMEM((CHUNK,), jnp.int32),
        pltpu.VMEM((CHUNK,), jnp.int32),
        pltpu.VMEM((CHUNK,), jnp.int32),
        pltpu.VMEM((128, H), jnp.float32),
        pltpu.VMEM((CHUNK, H), jnp.float32),
        pltpu.VMEM((CHUNK, H), jnp.float32),
        pltpu.VMEM_SHARED((N3, H), jnp.float32),
        pltpu.SemaphoreType.DMA,
        pltpu.SemaphoreType.DMA,
        pltpu.SemaphoreType.DMA,
        pltpu.SemaphoreType.DMA,
        pltpu.SemaphoreType.DMA,
        pltpu.SemaphoreType.DMA,
        pltpu.SemaphoreType.DMA,
        pltpu.SemaphoreType.DMA,
        pltpu.SemaphoreType.DMA,
        pltpu.SemaphoreType.DMA,
    ],
)
def _edge_kernel(h_hbm, tab_hbm, src_hbm, c_hbm, dst_hbm, out0, out1,
                 sv0, sv1, dv0, dv1, cv0, cv1, tab_v, hb0, hb1, aggr,
                 g0, g1, s0, s1, dI0, dI1, cI0, cI1, rI0, rI1):
    cid = lax.axis_index("c")
    sid = lax.axis_index("s")
    w = cid * NS + sid
    svs = (sv0, sv1)
    dvs = (dv0, dv1)
    cvs = (cv0, cv1)
    hbs = (hb0, hb1)
    gsem = (g0, g1)
    ssem = (s0, s1)
    dsem = (dI0, dI1)
    csem = (cI0, cI1)
    rsem = (rI0, rI1)
    ebase = w * NCHUNK * CHUNK

    pltpu.sync_copy(tab_hbm, tab_v)

    _zero_rows(hb0, CHUNK)
    base = sid * ROWS_PER_TILE
    nfull = ROWS_PER_TILE // CHUNK
    for k in range(nfull):
        pltpu.sync_copy(hb0, aggr.at[pl.ds(base + k * CHUNK, CHUNK)])
    rem = ROWS_PER_TILE - nfull * CHUNK
    if rem:
        pltpu.sync_copy(hb0.at[pl.ds(0, rem)],
                        aggr.at[pl.ds(base + nfull * CHUNK, rem)])
    plsc.subcore_barrier()

    def src_load(j, k):
        pltpu.async_copy(src_hbm.at[pl.ds(ebase + j * CHUNK, CHUNK)],
                         svs[k], rsem[k])

    def c_load(j, k):
        pltpu.async_copy(c_hbm.at[pl.ds(ebase + j * CHUNK, CHUNK)],
                         cvs[k], csem[k])

    def dst_load(j, k):
        pltpu.async_copy(dst_hbm.at[pl.ds(ebase + j * CHUNK, CHUNK)],
                         dvs[k], dsem[k])

    def gather(j, p):
        pltpu.async_copy(h_hbm.at[svs[p]], hbs[p], gsem[p])

    def wait_src(k):
        pltpu.make_async_copy(src_hbm.at[pl.ds(0, CHUNK)], svs[k],
                              rsem[k]).wait()

    def wait_c(k):
        pltpu.make_async_copy(c_hbm.at[pl.ds(0, CHUNK)], cvs[k],
                              csem[k]).wait()

    def wait_dst(k):
        pltpu.make_async_copy(dst_hbm.at[pl.ds(0, CHUNK)], dvs[k],
                              dsem[k]).wait()

    def wait_gather(p):
        pltpu.make_async_copy(h_hbm.at[pl.ds(0, CHUNK)], hbs[p],
                              gsem[p]).wait()

    def wait_scatter(p):
        pltpu.make_async_copy(hbs[p], aggr.at[pl.ds(0, CHUNK)],
                              ssem[p]).wait()

    pltpu.sync_copy(src_hbm.at[pl.ds(ebase, CHUNK)], sv0)
    c_load(0, 0)
    dst_load(0, 0)
    gather(0, 0)
    src_load(1, 1)
    c_load(1, 1)

    NT = NCHUNK // 2

    @pl.loop(0, NT)
    def _(t):
        for u in range(2):
            p = u
            pn = (u + 1) % 2
            if u == 0:
                @pl.when(t > 0)
                def _():
                    wait_scatter(pn)
            else:
                wait_scatter(pn)
            if u == 0:
                dst_load(2 * t + 1, pn)
            else:
                @pl.when(t < NT - 1)
                def _():
                    dst_load(2 * t + 2, pn)
            if u == 0:
                wait_src(pn)
                gather(2 * t + 1, pn)
            else:
                @pl.when(t < NT - 1)
                def _():
                    wait_src(pn)
                    gather(2 * t + 2, pn)
            wait_gather(p)
            wait_c(p)
            _msg_compute(hbs[p], tab_v, cvs[p])
            wait_dst(p)
            pltpu.async_copy(hbs[p], aggr.at[dvs[p]], ssem[p], add=True)
            if u == 0:
                @pl.when(t < NT - 1)
                def _():
                    src_load(2 * t + 2, p)
                    c_load(2 * t + 2, p)
            else:
                @pl.when(t < NT - 1)
                def _():
                    src_load(2 * t + 3, p)
                    c_load(2 * t + 3, p)

    wait_scatter(1)
    plsc.subcore_barrier()

    slab = pl.ds(base, ROWS_PER_TILE)

    @pl.when(cid == 0)
    def _():
        pltpu.sync_copy(aggr.at[slab], out0.at[slab])

    @pl.when(cid == 1)
    def _():
        pltpu.sync_copy(aggr.at[slab], out1.at[slab])


def _tab_body(bond_ref, o_ref):
    for i0 in range(5):
        b0 = bond_ref[0, pl.ds(i0, 1), :]
        for i1 in range(5):
            b01 = b0 + bond_ref[1, pl.ds(i1, 1), :]
            for i2 in range(5):
                o_ref[pl.ds(i0 * 25 + i1 * 5 + i2, 1), :] = (
                    b01 + bond_ref[2, pl.ds(i2, 1), :])
    o_ref[pl.ds(125, 1), :] = jnp.full((1, H), -1e30, jnp.float32)
    o_ref[pl.ds(126, 2), :] = jnp.zeros((2, H), jnp.float32)


def _tab_call(bond_emb):
    return pl.pallas_call(
        _tab_body,
        out_shape=jax.ShapeDtypeStruct((128, H), jnp.float32),
    )(bond_emb)


def _mlp_body(h_ref, a0_ref, a1_ref, epsb_ref, w1_ref, b1_ref, g1_ref,
              be1_ref, w2_ref, b2_ref, g2_ref, be2_ref, o_ref):
    z = h_ref[...] * epsb_ref[...] + (a0_ref[...] + a1_ref[...])
    z1 = jnp.dot(z, w1_ref[...], preferred_element_type=jnp.float32)
    z1 = z1 + b1_ref[...]
    m1 = jnp.mean(z1, axis=0, keepdims=True)
    v1 = jnp.mean((z1 - m1) ** 2, axis=0, keepdims=True)
    z1 = (z1 - m1) / jnp.sqrt(v1 + 1e-5) * g1_ref[...] + be1_ref[...]
    z1 = jnp.maximum(z1, 0.0)
    z2 = jnp.dot(z1, w2_ref[...], preferred_element_type=jnp.float32)
    z2 = z2 + b2_ref[...]
    m2 = jnp.mean(z2, axis=0, keepdims=True)
    v2 = jnp.mean((z2 - m2) ** 2, axis=0, keepdims=True)
    z2 = (z2 - m2) / jnp.sqrt(v2 + 1e-5) * g2_ref[...] + be2_ref[...]
    o_ref[...] = jnp.maximum(z2, 0.0)


def _mlp_call(h, a0, a1, epsb, w1, b1, g1, be1, w2, b2, g2, be2):
    return pl.pallas_call(
        _mlp_body,
        out_shape=jax.ShapeDtypeStruct((N, H), jnp.float32),
    )(h, a0, a1, epsb, w1, b1, g1, be1, w2, b2, g2, be2)


def _pool_body(h_ref, batch_ref, lw_ref, lb_ref, o_ref):
    onehot = (batch_ref[...] == lax.broadcasted_iota(jnp.int32, (N, G), 1)
              ).astype(jnp.float32)
    dn = (((0,), (0,)), ((), ()))
    sums = lax.dot_general(onehot, h_ref[...], dn,
                           preferred_element_type=jnp.float32)
    cnt = lax.dot_general(onehot, jnp.ones((N, 1), jnp.float32), dn,
                          preferred_element_type=jnp.float32)
    pooled = sums / jnp.maximum(cnt, 1.0)
    o_ref[...] = jnp.dot(pooled, lw_ref[...],
                         preferred_element_type=jnp.float32) + lb_ref[...]


def _pool_call(h, batch2d, lin_W, lin_b2d):
    return pl.pallas_call(
        _pool_body,
        out_shape=jax.ShapeDtypeStruct((G, OUT), jnp.float32),
    )(h, batch2d, lin_W, lin_b2d)


def kernel(x, edge_index, edge_attr, batch, atom_emb, bond_emb, eps,
           W1, b1, g1, beta1, W2, b2, g2, beta2, lin_W, lin_b):
    src = edge_index[0]
    dst = edge_index[1]
    c = edge_attr[:, 0] * 25 + edge_attr[:, 1] * 5 + edge_attr[:, 2]
    pad = E2 - E
    srcp = jnp.concatenate([src, jnp.zeros((pad,), jnp.int32)])
    dstp = jnp.concatenate([dst, jnp.zeros((pad,), jnp.int32)])
    cp = jnp.concatenate([c, jnp.full((pad,), 125, jnp.int32)])
    xt = jnp.pad(x, ((0, N2 - N), (0, 0))).T
    xt = (xt.reshape(9, NW, NODE_CHUNKS, CHUNK).transpose(1, 2, 0, 3)
          .reshape(-1))
    batch2d = batch.reshape(N, 1)

    tab = _tab_call(bond_emb)
    h = _encoder_kernel(xt, *(atom_emb[f] for f in range(9)))[:N]
    for l in range(L):
        a0, a1 = _edge_kernel(h, tab, srcp, cp, dstp)
        a0, a1 = a0[:N], a1[:N]
        epsb = jnp.broadcast_to(1.0 + eps[l], (1, H)).astype(jnp.float32)
        h = _mlp_call(h, a0, a1, epsb, W1[l], b1[l].reshape(1, 2 * H),
                      g1[l].reshape(1, 2 * H), beta1[l].reshape(1, 2 * H),
                      W2[l], b2[l].reshape(1, H), g2[l].reshape(1, H),
                      beta2[l].reshape(1, H))
    return _pool_call(h, batch2d, lin_W, lin_b.reshape(1, OUT))


# speedup vs baseline: 3.9968x; 1.0016x over previous
import functools

import jax
import jax.numpy as jnp
from jax import lax
from jax.experimental import pallas as pl
from jax.experimental.pallas import tpu as pltpu
from jax.experimental.pallas import tpu_sc as plsc

N = 10000
E = 320000
H = 128
OUT = 128
G = 256
L = 3

NC = 2
NS = 16
NW = NC * NS
CHUNK = 128
NCHUNK = 80
E2 = NW * NCHUNK * CHUNK

NODE_CHUNKS = 3
N2 = NW * NODE_CHUNKS * CHUNK
ROWS_PER_TILE = 632
N3 = NS * ROWS_PER_TILE

_mesh = plsc.VectorSubcoreMesh(core_axis_name="c", subcore_axis_name="s")


def _zero_rows(buf, nrows):
    zeros16 = jnp.zeros((1, 16), jnp.float32)

    @pl.loop(0, nrows)
    def _(r):
        for g in range(H // 16):
            buf[pl.ds(r, 1), pl.ds(g * 16, 16)] = zeros16


def _accum_rows(acc, src, nrows):
    @pl.loop(0, nrows)
    def _(r):
        for g in range(H // 16):
            sl = (pl.ds(r, 1), pl.ds(g * 16, 16))
            acc[sl] = acc[sl] + src[sl]


@functools.partial(
    pl.kernel,
    out_type=jax.ShapeDtypeStruct((N2, H), jnp.float32),
    mesh=_mesh,
    scratch_types=[
        pltpu.VMEM((NODE_CHUNKS * 9 * CHUNK,), jnp.int32),
        pltpu.VMEM((CHUNK, H), jnp.float32),
        pltpu.VMEM((CHUNK, H), jnp.float32),
        pltpu.VMEM((CHUNK, H), jnp.float32),
        pltpu.VMEM((CHUNK, H), jnp.float32),
        pltpu.SemaphoreType.DMA,
        pltpu.SemaphoreType.DMA,
        pltpu.SemaphoreType.DMA,
        pltpu.SemaphoreType.DMA,
        pltpu.SemaphoreType.DMA,
        pltpu.SemaphoreType.DMA,
    ],
)
def _encoder_kernel(xt_hbm, e0, e1, e2, e3, e4, e5, e6, e7, e8, o_hbm,
                    xv, acc0, acc1, tmp0, tmp1, aS0, aS1, tS0, tS1, oS0, oS1):
    embs = (e0, e1, e2, e3, e4, e5, e6, e7, e8)
    accs = (acc0, acc1)
    asem = (aS0, aS1)
    tmps = (tmp0, tmp1)
    tsem = (tS0, tS1)
    osem = (oS0, oS1)
    w = lax.axis_index("c") * NS + lax.axis_index("s")
    nidx = NODE_CHUNKS * 9 * CHUNK
    pltpu.sync_copy(xt_hbm.at[pl.ds(w * nidx, nidx)], xv)

    def idx(k, f):
        return xv.at[pl.ds((k * 9 + f) * CHUNK, CHUNK)]

    def wait_rows(buf, sem):
        pltpu.make_async_copy(e0.at[pl.ds(0, CHUNK)], buf, sem).wait()

    for k in range(NODE_CHUNKS):
        a = k % 2
        if k >= 2:
            pltpu.make_async_copy(accs[a], o_hbm.at[pl.ds(0, CHUNK)],
                                  osem[a]).wait()
        pltpu.async_copy(embs[0].at[idx(k, 0)], accs[a], asem[a])
        pltpu.async_copy(embs[1].at[idx(k, 1)], tmps[0], tsem[0])
        wait_rows(accs[a], asem[a])
        for f in range(1, 9):
            if f + 1 <= 8:
                pltpu.async_copy(embs[f + 1].at[idx(k, f + 1)],
                                 tmps[f % 2], tsem[f % 2])
            wait_rows(tmps[(f - 1) % 2], tsem[(f - 1) % 2])
            _accum_rows(accs[a], tmps[(f - 1) % 2], CHUNK)
        pltpu.async_copy(
            accs[a],
            o_hbm.at[pl.ds(w * (NODE_CHUNKS * CHUNK) + k * CHUNK, CHUNK)],
            osem[a])
    for k in range(max(0, NODE_CHUNKS - 2), NODE_CHUNKS):
        pltpu.make_async_copy(accs[k % 2], o_hbm.at[pl.ds(0, CHUNK)],
                              osem[k % 2]).wait()


def _msg_compute(hb, tab_v, cs):
    ng = H // 16

    @pl.loop(0, CHUNK // 16)
    def _(q):
        cvec = cs[pl.ds(q * 16, 16)]
        for k in range(16):
            cval = cvec[k]
            rsl = pl.ds(q * 16 + k, 1)
            trow = pl.ds(cval, 1)
            hv = [hb[rsl, pl.ds(g * 16, 16)] for g in range(ng)]
            tv = [tab_v[trow, pl.ds(g * 16, 16)] for g in range(ng)]
            res = [jnp.maximum(hv[g] + tv[g], 0.0) for g in range(ng)]
            for g in range(ng):
                hb[rsl, pl.ds(g * 16, 16)] = res[g]


@functools.partial(
    pl.kernel,
    out_type=[
        jax.ShapeDtypeStruct((N3, H), jnp.float32),
        jax.ShapeDtypeStruct((N3, H), jnp.float32),
    ],
    mesh=_mesh,
    scratch_types=[
        pltpu.VMEM((CHUNK,), jnp.int32),
        pltpu.VMEM((CHUNK,), jnp.int32),
        pltpu.VMEM((CHUNK // 2,), jnp.int32),
        pltpu.VMEM((CHUNK // 2,), jnp.int32),
        pltpu.VMEM((CHUNK // 2,), jnp.int32),
        pltpu.VMEM((CHUNK // 2,), jnp.int32),
        pltpu.VMEM((CHUNK,), jnp.int32),
        pltpu.VMEM((CHUNK,), jnp.int32),
        pltpu.VMEM((128, H), jnp.float32),
        pltpu.VMEM((CHUNK, H), jnp.float32),
        pltpu.VMEM((CHUNK, H), jnp.float32),
        pltpu.VMEM_SHARED((N3, H), jnp.float32),
        pltpu.SemaphoreType.DMA,
        pltpu.SemaphoreType.DMA,
        pltpu.SemaphoreType.DMA,
        pltpu.SemaphoreType.DMA,
        pltpu.SemaphoreType.DMA,
        pltpu.SemaphoreType.DMA,
        pltpu.SemaphoreType.DMA,
        pltpu.SemaphoreType.DMA,
        pltpu.SemaphoreType.DMA,
        pltpu.SemaphoreType.DMA,
    ],
)
def _edge_kernel(h_hbm, tab_hbm, src_hbm, c_hbm, dst_hbm, out0, out1,
                 sv0, sv1, dvA0, dvB0, dvA1, dvB1, cv0, cv1, tab_v,
                 hb0, hb1, aggr,
                 g0, g1, s0, s1, dI0, dI1, cI0, cI1, rI0, rI1):
    cid = lax.axis_index("c")
    sid = lax.axis_index("s")
    w = cid * NS + sid
    svs = (sv0, sv1)
    dvAs = (dvA0, dvA1)
    dvBs = (dvB0, dvB1)
    cvs = (cv0, cv1)
    hbs = (hb0, hb1)
    gsem = (g0, g1)
    ssem = (s0, s1)
    dsem = (dI0, dI1)
    csem = (cI0, cI1)
    rsem = (rI0, rI1)
    ebase = w * NCHUNK * CHUNK

    pltpu.sync_copy(tab_hbm, tab_v)

    _zero_rows(hb0, CHUNK)
    base = sid * ROWS_PER_TILE
    nfull = ROWS_PER_TILE // CHUNK
    for k in range(nfull):
        pltpu.sync_copy(hb0, aggr.at[pl.ds(base + k * CHUNK, CHUNK)])
    rem = ROWS_PER_TILE - nfull * CHUNK
    if rem:
        pltpu.sync_copy(hb0.at[pl.ds(0, rem)],
                        aggr.at[pl.ds(base + nfull * CHUNK, rem)])
    plsc.subcore_barrier()

    def src_load(j, k):
        pltpu.async_copy(src_hbm.at[pl.ds(ebase + j * CHUNK, CHUNK)],
                         svs[k], rsem[k])

    def c_load(j, k):
        pltpu.async_copy(c_hbm.at[pl.ds(ebase + j * CHUNK, CHUNK)],
                         cvs[k], csem[k])

    def dst_load(j, k):
        pltpu.async_copy(dst_hbm.at[pl.ds(ebase + j * CHUNK, CHUNK // 2)],
                         dvAs[k], dsem[k])
        pltpu.async_copy(
            dst_hbm.at[pl.ds(ebase + j * CHUNK + CHUNK // 2, CHUNK // 2)],
            dvBs[k], dsem[k])

    def gather(j, p):
        pltpu.async_copy(h_hbm.at[svs[p]], hbs[p], gsem[p])

    def wait_src(k):
        pltpu.make_async_copy(src_hbm.at[pl.ds(0, CHUNK)], svs[k],
                              rsem[k]).wait()

    def wait_c(k):
        pltpu.make_async_copy(c_hbm.at[pl.ds(0, CHUNK)], cvs[k],
                              csem[k]).wait()

    def wait_dst(k):
        pltpu.make_async_copy(dst_hbm.at[pl.ds(0, CHUNK // 2)], dvAs[k],
                              dsem[k]).wait()
        pltpu.make_async_copy(dst_hbm.at[pl.ds(0, CHUNK // 2)], dvBs[k],
                              dsem[k]).wait()

    def wait_gather(p):
        pltpu.make_async_copy(h_hbm.at[pl.ds(0, CHUNK)], hbs[p],
                              gsem[p]).wait()

    def wait_scatter(p):
        pltpu.make_async_copy(hbs[p], aggr.at[pl.ds(0, CHUNK)],
                              ssem[p]).wait()

    pltpu.sync_copy(src_hbm.at[pl.ds(ebase, CHUNK)], sv0)
    c_load(0, 0)
    dst_load(0, 0)
    gather(0, 0)
    src_load(1, 1)
    c_load(1, 1)

    NT = NCHUNK // 2

    @pl.loop(0, NT)
    def _(t):
        for u in range(2):
            p = u
            pn = (u + 1) % 2
            if u == 0:
                @pl.when(t > 0)
                def _():
                    wait_scatter(pn)
            else:
                wait_scatter(pn)
            if u == 0:
                dst_load(2 * t + 1, pn)
            else:
                @pl.when(t < NT - 1)
                def _():
                    dst_load(2 * t + 2, pn)
            if u == 0:
                wait_src(pn)
                gather(2 * t + 1, pn)
            else:
                @pl.when(t < NT - 1)
                def _():
                    wait_src(pn)
                    gather(2 * t + 2, pn)
            wait_gather(p)
            wait_c(p)
            _msg_compute(hbs[p], tab_v, cvs[p])
            wait_dst(p)
            pltpu.async_copy(hbs[p].at[pl.ds(0, CHUNK // 2)],
                             aggr.at[dvAs[p]], ssem[p], add=True)
            pltpu.async_copy(hbs[p].at[pl.ds(CHUNK // 2, CHUNK // 2)],
                             aggr.at[dvBs[p]], ssem[p], add=True)
            if u == 0:
                @pl.when(t < NT - 1)
                def _():
                    src_load(2 * t + 2, p)
                    c_load(2 * t + 2, p)
            else:
                @pl.when(t < NT - 1)
                def _():
                    src_load(2 * t + 3, p)
                    c_load(2 * t + 3, p)

    wait_scatter(1)
    plsc.subcore_barrier()

    slab = pl.ds(base, ROWS_PER_TILE)

    @pl.when(cid == 0)
    def _():
        pltpu.sync_copy(aggr.at[slab], out0.at[slab])

    @pl.when(cid == 1)
    def _():
        pltpu.sync_copy(aggr.at[slab], out1.at[slab])


def _tab_body(bond_ref, o_ref):
    for i0 in range(5):
        b0 = bond_ref[0, pl.ds(i0, 1), :]
        for i1 in range(5):
            b01 = b0 + bond_ref[1, pl.ds(i1, 1), :]
            for i2 in range(5):
                o_ref[pl.ds(i0 * 25 + i1 * 5 + i2, 1), :] = (
                    b01 + bond_ref[2, pl.ds(i2, 1), :])
    o_ref[pl.ds(125, 1), :] = jnp.full((1, H), -1e30, jnp.float32)
    o_ref[pl.ds(126, 2), :] = jnp.zeros((2, H), jnp.float32)


def _tab_call(bond_emb):
    return pl.pallas_call(
        _tab_body,
        out_shape=jax.ShapeDtypeStruct((128, H), jnp.float32),
    )(bond_emb)


def _mlp_body(h_ref, a0_ref, a1_ref, epsb_ref, w1_ref, b1_ref, g1_ref,
              be1_ref, w2_ref, b2_ref, g2_ref, be2_ref, o_ref):
    z = h_ref[...] * epsb_ref[...] + (a0_ref[...] + a1_ref[...])
    z1 = jnp.dot(z, w1_ref[...], preferred_element_type=jnp.float32)
    z1 = z1 + b1_ref[...]
    m1 = jnp.mean(z1, axis=0, keepdims=True)
    v1 = jnp.mean((z1 - m1) ** 2, axis=0, keepdims=True)
    z1 = (z1 - m1) / jnp.sqrt(v1 + 1e-5) * g1_ref[...] + be1_ref[...]
    z1 = jnp.maximum(z1, 0.0)
    z2 = jnp.dot(z1, w2_ref[...], preferred_element_type=jnp.float32)
    z2 = z2 + b2_ref[...]
    m2 = jnp.mean(z2, axis=0, keepdims=True)
    v2 = jnp.mean((z2 - m2) ** 2, axis=0, keepdims=True)
    z2 = (z2 - m2) / jnp.sqrt(v2 + 1e-5) * g2_ref[...] + be2_ref[...]
    o_ref[...] = jnp.maximum(z2, 0.0)


def _mlp_call(h, a0, a1, epsb, w1, b1, g1, be1, w2, b2, g2, be2):
    return pl.pallas_call(
        _mlp_body,
        out_shape=jax.ShapeDtypeStruct((N, H), jnp.float32),
    )(h, a0, a1, epsb, w1, b1, g1, be1, w2, b2, g2, be2)


def _pool_body(h_ref, batch_ref, lw_ref, lb_ref, o_ref):
    onehot = (batch_ref[...] == lax.broadcasted_iota(jnp.int32, (N, G), 1)
              ).astype(jnp.float32)
    dn = (((0,), (0,)), ((), ()))
    sums = lax.dot_general(onehot, h_ref[...], dn,
                           preferred_element_type=jnp.float32)
    cnt = lax.dot_general(onehot, jnp.ones((N, 1), jnp.float32), dn,
                          preferred_element_type=jnp.float32)
    pooled = sums / jnp.maximum(cnt, 1.0)
    o_ref[...] = jnp.dot(pooled, lw_ref[...],
                         preferred_element_type=jnp.float32) + lb_ref[...]


def _pool_call(h, batch2d, lin_W, lin_b2d):
    return pl.pallas_call(
        _pool_body,
        out_shape=jax.ShapeDtypeStruct((G, OUT), jnp.float32),
    )(h, batch2d, lin_W, lin_b2d)


def kernel(x, edge_index, edge_attr, batch, atom_emb, bond_emb, eps,
           W1, b1, g1, beta1, W2, b2, g2, beta2, lin_W, lin_b):
    src = edge_index[0]
    dst = edge_index[1]
    c = edge_attr[:, 0] * 25 + edge_attr[:, 1] * 5 + edge_attr[:, 2]
    pad = E2 - E
    srcp = jnp.concatenate([src, jnp.zeros((pad,), jnp.int32)])
    dstp = jnp.concatenate([dst, jnp.zeros((pad,), jnp.int32)])
    cp = jnp.concatenate([c, jnp.full((pad,), 125, jnp.int32)])
    xt = jnp.pad(x, ((0, N2 - N), (0, 0))).T
    xt = (xt.reshape(9, NW, NODE_CHUNKS, CHUNK).transpose(1, 2, 0, 3)
          .reshape(-1))
    batch2d = batch.reshape(N, 1)

    tab = _tab_call(bond_emb)
    h = _encoder_kernel(xt, *(atom_emb[f] for f in range(9)))[:N]
    for l in range(L):
        a0, a1 = _edge_kernel(h, tab, srcp, cp, dstp)
        a0, a1 = a0[:N], a1[:N]
        epsb = jnp.broadcast_to(1.0 + eps[l], (1, H)).astype(jnp.float32)
        h = _mlp_call(h, a0, a1, epsb, W1[l], b1[l].reshape(1, 2 * H),
                      g1[l].reshape(1, 2 * H), beta1[l].reshape(1, 2 * H),
                      W2[l], b2[l].reshape(1, H), g2[l].reshape(1, H),
                      beta2[l].reshape(1, H))
    return _pool_call(h, batch2d, lin_W, lin_b.reshape(1, OUT))
